# baseline (speedup 1.0000x reference)
import functools

import jax
import jax.numpy as jnp
from jax import lax
from jax.experimental import pallas as pl
from jax.experimental.pallas import tpu as pltpu
from jax.experimental.pallas import tpu_sc as plsc

N = 10000
NPAD = 10240
E = 320000
D_IN = 128
D = 64
BN_RS = 1.0 / (1.0 + 1e-5) ** 0.5

NC = 2
NS = 16
NW = NC * NS
CHUNK = 128
NITER = 80
EPT = CHUNK * NITER
EPAD = NW * EPT
RPT = NPAD // NS
DEGW = 16

BLK = 512
NB = NPAD // BLK

@functools.lru_cache(maxsize=None)
def _sc_mesh():
    return plsc.VectorSubcoreMesh(
        core_axis_name="c", subcore_axis_name="s",
        num_cores=NC, num_subcores=NS)


def _deg_body(dst_hbm, ones_hbm, zeros_hbm, out0_hbm, out1_hbm,
              idx_v, ones_v, zrow_v, acc_sh):
    c = lax.axis_index("c")
    s = lax.axis_index("s")
    w = s * NC + c
    pltpu.sync_copy(zeros_hbm, zrow_v)
    pltpu.sync_copy(zrow_v, acc_sh.at[pl.ds(s * RPT, RPT)])
    pltpu.sync_copy(ones_hbm, ones_v)
    pltpu.sync_copy(dst_hbm.at[w], idx_v)
    plsc.subcore_barrier()

    def body(i, carry):
        pltpu.sync_copy(ones_v, acc_sh.at[idx_v.at[i]], add=True)
        return carry

    lax.fori_loop(0, NITER, body, 0)
    plsc.subcore_barrier()

    @pl.when(c == 0)
    def _():
        pltpu.sync_copy(acc_sh.at[pl.ds(s * RPT, RPT)],
                        out0_hbm.at[pl.ds(s * RPT, RPT)])

    @pl.when(c == 1)
    def _():
        pltpu.sync_copy(acc_sh.at[pl.ds(s * RPT, RPT)],
                        out1_hbm.at[pl.ds(s * RPT, RPT)])


@functools.lru_cache(maxsize=None)
def _deg_kernel_build():
    return pl.kernel(
        _deg_body,
        out_type=[jax.ShapeDtypeStruct((NPAD, DEGW), jnp.float32),
                  jax.ShapeDtypeStruct((NPAD, DEGW), jnp.float32)],
        mesh=_sc_mesh(),
        scratch_types=[
            pltpu.VMEM((NITER, CHUNK), jnp.int32),
            pltpu.VMEM((CHUNK, DEGW), jnp.float32),
            pltpu.VMEM((RPT, DEGW), jnp.float32),
            pltpu.VMEM_SHARED((NPAD, DEGW), jnp.float32),
        ],
        compiler_params=pltpu.CompilerParams(use_tc_tiling_on_sc=False),
    )


def _deg_kernel(*args):
    return _deg_kernel_build()(*args)


NBUF = 4


def _agg_body(h_hbm, src_hbm, dst_hbm, zeros_hbm, out0_hbm, out1_hbm,
              idxs_v, idxd_v, r0, r1, r2, r3, zrow_v, acc_sh,
              g0, g1, g2, g3, s0, s1, s2, s3):
    rows = (r0, r1, r2, r3)
    sem_g = (g0, g1, g2, g3)
    sem_s = (s0, s1, s2, s3)
    c = lax.axis_index("c")
    s = lax.axis_index("s")
    w = s * NC + c
    pltpu.sync_copy(zeros_hbm, zrow_v)
    pltpu.sync_copy(zrow_v, acc_sh.at[pl.ds(s * RPT, RPT)])
    pltpu.sync_copy(src_hbm.at[w], idxs_v)
    pltpu.sync_copy(dst_hbm.at[w], idxd_v)
    plsc.subcore_barrier()

    def _wait(buf, sem):
        pltpu.make_async_copy(h_hbm.at[pl.ds(0, CHUNK)], buf, sem).wait()

    for k in range(NBUF):
        pltpu.async_copy(h_hbm.at[idxs_v.at[k]], rows[k], sem_g[k])

    def body(t, carry):
        j = NBUF * t
        for k in range(NBUF):
            _wait(rows[k], sem_g[k])
            pltpu.async_copy(rows[k], acc_sh.at[idxd_v.at[j + k]],
                             sem_s[k], add=True)
        for k in range(NBUF):
            @pl.when(j + k + NBUF < NITER)
            def _(k=k):
                _wait(rows[k], sem_s[k])
                pltpu.async_copy(h_hbm.at[idxs_v.at[j + k + NBUF]],
                                 rows[k], sem_g[k])

        return carry

    lax.fori_loop(0, NITER // NBUF, body, 0)
    for k in range(NBUF):
        _wait(rows[k], sem_s[k])
    plsc.subcore_barrier()

    @pl.when(c == 0)
    def _():
        pltpu.sync_copy(acc_sh.at[pl.ds(s * RPT, RPT)],
                        out0_hbm.at[pl.ds(s * RPT, RPT)])

    @pl.when(c == 1)
    def _():
        pltpu.sync_copy(acc_sh.at[pl.ds(s * RPT, RPT)],
                        out1_hbm.at[pl.ds(s * RPT, RPT)])


@functools.lru_cache(maxsize=None)
def _agg_kernel_build():
    return pl.kernel(
        _agg_body,
        out_type=[jax.ShapeDtypeStruct((NPAD, D), jnp.bfloat16),
                  jax.ShapeDtypeStruct((NPAD, D), jnp.bfloat16)],
        mesh=_sc_mesh(),
        scratch_types=[
            pltpu.VMEM((NITER, CHUNK), jnp.int32),
            pltpu.VMEM((NITER, CHUNK), jnp.int32),
        ] + [pltpu.VMEM((CHUNK, D), jnp.bfloat16)] * NBUF + [
            pltpu.VMEM((RPT, D), jnp.bfloat16),
            pltpu.VMEM_SHARED((NPAD, D), jnp.bfloat16),
        ] + [pltpu.SemaphoreType.DMA] * (2 * NBUF),
        compiler_params=pltpu.CompilerParams(use_tc_tiling_on_sc=False),
    )


def _agg_kernel(*args):
    return _agg_kernel_build()(*args)


def _mm1_body(x_ref, w_ref, h_ref):
    h_ref[...] = jnp.dot(x_ref[...], w_ref[...],
                         preferred_element_type=jnp.float32)


def _prep1_body(h_ref, d0_ref, d1_ref, b_ref, hhat_ref, self_ref):
    h = h_ref[...]
    deg = d0_ref[:, 0:1] + d1_ref[:, 0:1] + 1.0
    dinv = lax.rsqrt(deg)
    hhat_ref[...] = (h * dinv).astype(jnp.bfloat16)
    self_ref[...] = h * (dinv * dinv) + b_ref[...]


def _mid_body(a0_ref, a1_ref, self_ref, d0_ref, d1_ref, g_ref, be_ref,
              w_ref, b_ref, selfo_ref, hhat_ref):
    deg = d0_ref[:, 0:1] + d1_ref[:, 0:1] + 1.0
    dinv = lax.rsqrt(deg)
    agg = a0_ref[...].astype(jnp.float32) + a1_ref[...].astype(jnp.float32)
    out = self_ref[...] + dinv * agg
    t = jnp.maximum(out * (g_ref[...] * BN_RS) + be_ref[...], 0.0)
    h = jnp.dot(t, w_ref[...], preferred_element_type=jnp.float32)
    selfo_ref[...] = h * (dinv * dinv) + b_ref[...]
    hhat_ref[...] = (h * dinv).astype(jnp.bfloat16)


def _final_body(a0_ref, a1_ref, self_ref, d0_ref, d1_ref, g_ref, be_ref,
                fc1w_ref, fc1b_ref, fc2w_ref, fc2b_ref, out_ref, acc):
    i = pl.program_id(0)
    deg = d0_ref[:, 0:1] + d1_ref[:, 0:1] + 1.0
    dinv = lax.rsqrt(deg)
    agg = a0_ref[...].astype(jnp.float32) + a1_ref[...].astype(jnp.float32)
    out = self_ref[...] + dinv * agg
    t = jnp.maximum(out * (g_ref[...] * BN_RS) + be_ref[...], 0.0)
    rows = i * BLK + lax.broadcasted_iota(jnp.int32, (BLK, 1), 0)
    t = jnp.where(rows < N, t, 0.0)
    part = jnp.sum(t, axis=0, keepdims=True)

    @pl.when(i == 0)
    def _():
        acc[...] = jnp.zeros_like(acc)

    acc[...] += jnp.broadcast_to(part, acc.shape)

    @pl.when(i == NB - 1)
    def _():
        mean = acc[0:1, :] * (1.0 / N)
        z = jnp.maximum(
            jnp.dot(mean, fc1w_ref[...],
                    preferred_element_type=jnp.float32) + fc1b_ref[...], 0.0)
        logits = jnp.dot(z, fc2w_ref[...],
                         preferred_element_type=jnp.float32) + fc2b_ref[...]
        out_ref[...] = jnp.broadcast_to(logits, out_ref.shape)


def _row_spec(width):
    return pl.BlockSpec((BLK, width), lambda i: (i, 0))


def _fix_spec(shape):
    return pl.BlockSpec(shape, lambda i: tuple(0 for _ in shape))


_mm1_call = pl.pallas_call(
    _mm1_body,
    grid=(NB,),
    in_specs=[_row_spec(D_IN), _fix_spec((D_IN, D))],
    out_specs=_row_spec(D),
    out_shape=jax.ShapeDtypeStruct((NPAD, D), jnp.float32),
)

_prep1_call = pl.pallas_call(
    _prep1_body,
    grid=(NB,),
    in_specs=[_row_spec(D), _row_spec(DEGW), _row_spec(DEGW),
              _fix_spec((1, D))],
    out_specs=[_row_spec(D), _row_spec(D)],
    out_shape=[jax.ShapeDtypeStruct((NPAD, D), jnp.bfloat16),
               jax.ShapeDtypeStruct((NPAD, D), jnp.float32)],
)

_mid_call = pl.pallas_call(
    _mid_body,
    grid=(NB,),
    in_specs=[_row_spec(D), _row_spec(D), _row_spec(D),
              _row_spec(DEGW), _row_spec(DEGW),
              _fix_spec((1, D)), _fix_spec((1, D)),
              _fix_spec((D, D)), _fix_spec((1, D))],
    out_specs=[_row_spec(D), _row_spec(D)],
    out_shape=[jax.ShapeDtypeStruct((NPAD, D), jnp.float32),
               jax.ShapeDtypeStruct((NPAD, D), jnp.bfloat16)],
)

_final_call = pl.pallas_call(
    _final_body,
    grid=(NB,),
    in_specs=[_row_spec(D), _row_spec(D), _row_spec(D),
              _row_spec(DEGW), _row_spec(DEGW),
              _fix_spec((1, D)), _fix_spec((1, D)),
              _fix_spec((D, 128)), _fix_spec((1, 128)),
              _fix_spec((128, 128)), _fix_spec((1, 128))],
    out_specs=_fix_spec((8, 128)),
    out_shape=jax.ShapeDtypeStruct((8, 128), jnp.float32),
    scratch_shapes=[pltpu.VMEM((8, D), jnp.float32)],
    compiler_params=pltpu.CompilerParams(
        dimension_semantics=("arbitrary",)),
)


def kernel(x, edge_index, W1, b1, g1, be1, W2, b2, g2, be2, W3, b3, g3, be3,
           fc1_w, fc1_b, fc2_w, fc2_b):
    pad_idx = N + jnp.broadcast_to(
        jnp.arange(NPAD - N, dtype=jnp.int32),
        ((EPAD - E) // (NPAD - N), NPAD - N)).reshape(-1)
    src = jnp.concatenate([edge_index[0].astype(jnp.int32), pad_idx])
    dst = jnp.concatenate([edge_index[1].astype(jnp.int32), pad_idx])
    src = src.reshape(NW, NITER, CHUNK)
    dst = dst.reshape(NW, NITER, CHUNK)
    x_pad = jnp.pad(x, ((0, NPAD - N), (0, 0)))

    ones_deg = jnp.ones((CHUNK, DEGW), jnp.float32)
    zeros_deg = jnp.zeros((RPT, DEGW), jnp.float32)
    zeros_agg = jnp.zeros((RPT, D), jnp.bfloat16)

    h1 = _mm1_call(x_pad, W1)
    deg0, deg1 = _deg_kernel(dst, ones_deg, zeros_deg)

    hhat, selfc = _prep1_call(h1, deg0, deg1, b1.reshape(1, D))

    for g, be, w, b in ((g1, be1, W2, b2), (g2, be2, W3, b3)):
        a0, a1 = _agg_kernel(hhat, src, dst, zeros_agg)
        selfc, hhat = _mid_call(
            a0, a1, selfc, deg0, deg1, g.reshape(1, D), be.reshape(1, D),
            w, b.reshape(1, D))

    a0, a1 = _agg_kernel(hhat, src, dst, zeros_agg)

    fc1w_pad = jnp.pad(fc1_w, ((0, 0), (0, 128 - fc1_w.shape[1])))
    fc1b_pad = jnp.pad(fc1_b, (0, 128 - fc1_b.shape[0])).reshape(1, 128)
    fc2w_pad = jnp.pad(fc2_w, ((0, 128 - fc2_w.shape[0]),
                               (0, 128 - fc2_w.shape[1])))
    fc2b_pad = jnp.pad(fc2_b, (0, 128 - fc2_b.shape[0])).reshape(1, 128)

    out = _final_call(a0, a1, selfc, deg0, deg1, g3.reshape(1, D),
                      be3.reshape(1, D), fc1w_pad, fc1b_pad,
                      fc2w_pad, fc2b_pad)
    return out[0:1, 0:3]


# speedup vs baseline: 1.1181x; 1.1181x over previous
import functools

import jax
import jax.numpy as jnp
from jax import lax
from jax.experimental import pallas as pl
from jax.experimental.pallas import tpu as pltpu
from jax.experimental.pallas import tpu_sc as plsc

N = 10000
NPAD = 10240
E = 320000
D_IN = 128
D = 64
BN_RS = 1.0 / (1.0 + 1e-5) ** 0.5

NC = 2
NS = 16
NW = NC * NS
CHUNK = 128
NITER = 80
EPT = CHUNK * NITER
EPAD = NW * EPT
RPT = NPAD // NS
DEGW = 16

BLK = 1024
NB = NPAD // BLK

@functools.lru_cache(maxsize=None)
def _sc_mesh():
    return plsc.VectorSubcoreMesh(
        core_axis_name="c", subcore_axis_name="s",
        num_cores=NC, num_subcores=NS)


def _deg_body(dst_hbm, ones_hbm, zeros_hbm, out0_hbm, out1_hbm,
              idx_v, ones_v, zrow_v, acc_sh):
    c = lax.axis_index("c")
    s = lax.axis_index("s")
    w = s * NC + c
    pltpu.sync_copy(zeros_hbm, zrow_v)
    pltpu.sync_copy(zrow_v, acc_sh.at[pl.ds(s * RPT, RPT)])
    pltpu.sync_copy(ones_hbm, ones_v)
    pltpu.sync_copy(dst_hbm.at[w], idx_v)
    plsc.subcore_barrier()

    def body(i, carry):
        pltpu.sync_copy(ones_v, acc_sh.at[idx_v.at[i]], add=True)
        return carry

    lax.fori_loop(0, NITER, body, 0)
    plsc.subcore_barrier()

    @pl.when(c == 0)
    def _():
        pltpu.sync_copy(acc_sh.at[pl.ds(s * RPT, RPT)],
                        out0_hbm.at[pl.ds(s * RPT, RPT)])

    @pl.when(c == 1)
    def _():
        pltpu.sync_copy(acc_sh.at[pl.ds(s * RPT, RPT)],
                        out1_hbm.at[pl.ds(s * RPT, RPT)])


@functools.lru_cache(maxsize=None)
def _deg_kernel_build():
    return pl.kernel(
        _deg_body,
        out_type=[jax.ShapeDtypeStruct((NPAD, DEGW), jnp.float32),
                  jax.ShapeDtypeStruct((NPAD, DEGW), jnp.float32)],
        mesh=_sc_mesh(),
        scratch_types=[
            pltpu.VMEM((NITER, CHUNK), jnp.int32),
            pltpu.VMEM((CHUNK, DEGW), jnp.float32),
            pltpu.VMEM((RPT, DEGW), jnp.float32),
            pltpu.VMEM_SHARED((NPAD, DEGW), jnp.float32),
        ],
        compiler_params=pltpu.CompilerParams(use_tc_tiling_on_sc=False),
    )


def _deg_kernel(*args):
    return _deg_kernel_build()(*args)


NBUF = 8


def _agg_body(h_hbm, src_hbm, dst_hbm, zeros_hbm, out0_hbm, out1_hbm,
              idxs_v, idxd_v, r0, r1, r2, r3, r4, r5, r6, r7, zrow_v, acc_sh,
              g0, g1, g2, g3, g4, g5, g6, g7,
              s0, s1, s2, s3, s4, s5, s6, s7):
    rows = (r0, r1, r2, r3, r4, r5, r6, r7)
    sem_g = (g0, g1, g2, g3, g4, g5, g6, g7)
    sem_s = (s0, s1, s2, s3, s4, s5, s6, s7)
    c = lax.axis_index("c")
    s = lax.axis_index("s")
    w = s * NC + c
    pltpu.sync_copy(zeros_hbm, zrow_v)
    pltpu.sync_copy(zrow_v, acc_sh.at[pl.ds(s * RPT, RPT)])
    pltpu.sync_copy(src_hbm.at[w], idxs_v)
    pltpu.sync_copy(dst_hbm.at[w], idxd_v)
    plsc.subcore_barrier()

    def _wait(buf, sem):
        pltpu.make_async_copy(h_hbm.at[pl.ds(0, CHUNK)], buf, sem).wait()

    for k in range(NBUF):
        pltpu.async_copy(h_hbm.at[idxs_v.at[k]], rows[k], sem_g[k])

    def body(t, carry):
        j = NBUF * t
        for k in range(NBUF):
            _wait(rows[k], sem_g[k])
            pltpu.async_copy(rows[k], acc_sh.at[idxd_v.at[j + k]],
                             sem_s[k], add=True)
        for k in range(NBUF):
            @pl.when(j + k + NBUF < NITER)
            def _(k=k):
                _wait(rows[k], sem_s[k])
                pltpu.async_copy(h_hbm.at[idxs_v.at[j + k + NBUF]],
                                 rows[k], sem_g[k])

        return carry

    lax.fori_loop(0, NITER // NBUF, body, 0)
    for k in range(NBUF):
        _wait(rows[k], sem_s[k])
    plsc.subcore_barrier()

    @pl.when(c == 0)
    def _():
        pltpu.sync_copy(acc_sh.at[pl.ds(s * RPT, RPT)],
                        out0_hbm.at[pl.ds(s * RPT, RPT)])

    @pl.when(c == 1)
    def _():
        pltpu.sync_copy(acc_sh.at[pl.ds(s * RPT, RPT)],
                        out1_hbm.at[pl.ds(s * RPT, RPT)])


@functools.lru_cache(maxsize=None)
def _agg_kernel_build():
    return pl.kernel(
        _agg_body,
        out_type=[jax.ShapeDtypeStruct((NPAD, D), jnp.bfloat16),
                  jax.ShapeDtypeStruct((NPAD, D), jnp.bfloat16)],
        mesh=_sc_mesh(),
        scratch_types=[
            pltpu.VMEM((NITER, CHUNK), jnp.int32),
            pltpu.VMEM((NITER, CHUNK), jnp.int32),
        ] + [pltpu.VMEM((CHUNK, D), jnp.bfloat16)] * NBUF + [
            pltpu.VMEM((RPT, D), jnp.bfloat16),
            pltpu.VMEM_SHARED((NPAD, D), jnp.bfloat16),
        ] + [pltpu.SemaphoreType.DMA] * (2 * NBUF),
        compiler_params=pltpu.CompilerParams(use_tc_tiling_on_sc=False),
    )


def _agg_kernel(*args):
    return _agg_kernel_build()(*args)


def _mm1_body(x_ref, w_ref, h_ref):
    h_ref[...] = jnp.dot(x_ref[...], w_ref[...],
                         preferred_element_type=jnp.float32)


def _prep1_body(h_ref, d0_ref, d1_ref, b_ref, hhat_ref, self_ref):
    h = h_ref[...]
    deg = d0_ref[:, 0:1] + d1_ref[:, 0:1] + 1.0
    dinv = lax.rsqrt(deg)
    hhat_ref[...] = (h * dinv).astype(jnp.bfloat16)
    self_ref[...] = h * (dinv * dinv) + b_ref[...]


def _mid_body(a0_ref, a1_ref, self_ref, d0_ref, d1_ref, g_ref, be_ref,
              w_ref, b_ref, selfo_ref, hhat_ref):
    deg = d0_ref[:, 0:1] + d1_ref[:, 0:1] + 1.0
    dinv = lax.rsqrt(deg)
    agg = a0_ref[...].astype(jnp.float32) + a1_ref[...].astype(jnp.float32)
    out = self_ref[...] + dinv * agg
    t = jnp.maximum(out * (g_ref[...] * BN_RS) + be_ref[...], 0.0)
    h = jnp.dot(t, w_ref[...], preferred_element_type=jnp.float32)
    selfo_ref[...] = h * (dinv * dinv) + b_ref[...]
    hhat_ref[...] = (h * dinv).astype(jnp.bfloat16)


def _final_body(a0_ref, a1_ref, self_ref, d0_ref, d1_ref, g_ref, be_ref,
                fc1w_ref, fc1b_ref, fc2w_ref, fc2b_ref, out_ref, acc):
    i = pl.program_id(0)
    deg = d0_ref[:, 0:1] + d1_ref[:, 0:1] + 1.0
    dinv = lax.rsqrt(deg)
    agg = a0_ref[...].astype(jnp.float32) + a1_ref[...].astype(jnp.float32)
    out = self_ref[...] + dinv * agg
    t = jnp.maximum(out * (g_ref[...] * BN_RS) + be_ref[...], 0.0)
    rows = i * BLK + lax.broadcasted_iota(jnp.int32, (BLK, 1), 0)
    t = jnp.where(rows < N, t, 0.0)
    part = jnp.sum(t, axis=0, keepdims=True)

    @pl.when(i == 0)
    def _():
        acc[...] = jnp.zeros_like(acc)

    acc[...] += jnp.broadcast_to(part, acc.shape)

    @pl.when(i == NB - 1)
    def _():
        mean = acc[0:1, :] * (1.0 / N)
        z = jnp.maximum(
            jnp.dot(mean, fc1w_ref[...],
                    preferred_element_type=jnp.float32) + fc1b_ref[...], 0.0)
        logits = jnp.dot(z, fc2w_ref[...],
                         preferred_element_type=jnp.float32) + fc2b_ref[...]
        out_ref[...] = jnp.broadcast_to(logits, out_ref.shape)


def _row_spec(width):
    return pl.BlockSpec((BLK, width), lambda i: (i, 0))


def _fix_spec(shape):
    return pl.BlockSpec(shape, lambda i: tuple(0 for _ in shape))


_mm1_call = pl.pallas_call(
    _mm1_body,
    grid=(NB,),
    in_specs=[_row_spec(D_IN), _fix_spec((D_IN, D))],
    out_specs=_row_spec(D),
    out_shape=jax.ShapeDtypeStruct((NPAD, D), jnp.float32),
)

_prep1_call = pl.pallas_call(
    _prep1_body,
    grid=(NB,),
    in_specs=[_row_spec(D), _row_spec(DEGW), _row_spec(DEGW),
              _fix_spec((1, D))],
    out_specs=[_row_spec(D), _row_spec(D)],
    out_shape=[jax.ShapeDtypeStruct((NPAD, D), jnp.bfloat16),
               jax.ShapeDtypeStruct((NPAD, D), jnp.float32)],
)

_mid_call = pl.pallas_call(
    _mid_body,
    grid=(NB,),
    in_specs=[_row_spec(D), _row_spec(D), _row_spec(D),
              _row_spec(DEGW), _row_spec(DEGW),
              _fix_spec((1, D)), _fix_spec((1, D)),
              _fix_spec((D, D)), _fix_spec((1, D))],
    out_specs=[_row_spec(D), _row_spec(D)],
    out_shape=[jax.ShapeDtypeStruct((NPAD, D), jnp.float32),
               jax.ShapeDtypeStruct((NPAD, D), jnp.bfloat16)],
)

_final_call = pl.pallas_call(
    _final_body,
    grid=(NB,),
    in_specs=[_row_spec(D), _row_spec(D), _row_spec(D),
              _row_spec(DEGW), _row_spec(DEGW),
              _fix_spec((1, D)), _fix_spec((1, D)),
              _fix_spec((D, 128)), _fix_spec((1, 128)),
              _fix_spec((128, 128)), _fix_spec((1, 128))],
    out_specs=_fix_spec((8, 128)),
    out_shape=jax.ShapeDtypeStruct((8, 128), jnp.float32),
    scratch_shapes=[pltpu.VMEM((8, D), jnp.float32)],
    compiler_params=pltpu.CompilerParams(
        dimension_semantics=("arbitrary",)),
)


def kernel(x, edge_index, W1, b1, g1, be1, W2, b2, g2, be2, W3, b3, g3, be3,
           fc1_w, fc1_b, fc2_w, fc2_b):
    pad_idx = N + jnp.broadcast_to(
        jnp.arange(NPAD - N, dtype=jnp.int32),
        ((EPAD - E) // (NPAD - N), NPAD - N)).reshape(-1)
    src = jnp.concatenate([edge_index[0].astype(jnp.int32), pad_idx])
    dst = jnp.concatenate([edge_index[1].astype(jnp.int32), pad_idx])
    src = src.reshape(NW, NITER, CHUNK)
    dst = dst.reshape(NW, NITER, CHUNK)
    x_pad = jnp.pad(x, ((0, NPAD - N), (0, 0)))

    ones_deg = jnp.ones((CHUNK, DEGW), jnp.float32)
    zeros_deg = jnp.zeros((RPT, DEGW), jnp.float32)
    zeros_agg = jnp.zeros((RPT, D), jnp.bfloat16)

    h1 = _mm1_call(x_pad, W1)
    deg0, deg1 = _deg_kernel(dst, ones_deg, zeros_deg)

    hhat, selfc = _prep1_call(h1, deg0, deg1, b1.reshape(1, D))

    for g, be, w, b in ((g1, be1, W2, b2), (g2, be2, W3, b3)):
        a0, a1 = _agg_kernel(hhat, src, dst, zeros_agg)
        selfc, hhat = _mid_call(
            a0, a1, selfc, deg0, deg1, g.reshape(1, D), be.reshape(1, D),
            w, b.reshape(1, D))

    a0, a1 = _agg_kernel(hhat, src, dst, zeros_agg)

    fc1w_pad = jnp.pad(fc1_w, ((0, 0), (0, 128 - fc1_w.shape[1])))
    fc1b_pad = jnp.pad(fc1_b, (0, 128 - fc1_b.shape[0])).reshape(1, 128)
    fc2w_pad = jnp.pad(fc2_w, ((0, 128 - fc2_w.shape[0]),
                               (0, 128 - fc2_w.shape[1])))
    fc2b_pad = jnp.pad(fc2_b, (0, 128 - fc2_b.shape[0])).reshape(1, 128)

    out = _final_call(a0, a1, selfc, deg0, deg1, g3.reshape(1, D),
                      be3.reshape(1, D), fc1w_pad, fc1b_pad,
                      fc2w_pad, fc2b_pad)
    return out[0:1, 0:3]
